# baseline (speedup 1.0000x reference)
import functools

import jax
import jax.numpy as jnp
from jax import lax
from jax.experimental import pallas as pl
from jax.experimental.pallas import tpu as pltpu
from jax.experimental.pallas import tpu_sc as plsc

NC = 2
NS = 16
L = 16
NW = NC * NS

BATCH = 4096
SEQ = 200
FEAT = 64
N = BATCH * SEQ
B_PER_W = N // NW
CS = 1024
N_STREAMS = CS // 128
N_CHUNKS = B_PER_W // CS


def _body(table, toks, pos, out, idx_v, rows_v, pos_v, sem):
    wid = lax.axis_index("s") * NC + lax.axis_index("c")
    base = wid * B_PER_W
    pltpu.sync_copy(pos, pos_v)

    def chunk(c, _):
        row0 = pl.multiple_of(base + c * CS, CS)
        pltpu.sync_copy(
            toks.at[pl.ds(pl.multiple_of(row0 // 128, 8), N_STREAMS)], idx_v)
        for j in range(N_STREAMS):
            pltpu.async_copy(
                table.at[idx_v.at[j]],
                rows_v.at[pl.ds(j * 128, 128)],
                sem,
            )
        for j in range(N_STREAMS):
            pltpu.make_async_copy(
                table.at[idx_v.at[j]],
                rows_v.at[pl.ds(j * 128, 128)],
                sem,
            ).wait()

        def add_pos(r, p):
            for j in range(FEAT // L):
                sl = pl.ds(j * L, L)
                rows_v[r, sl] = rows_v[r, sl] + pos_v[p, sl]
            return lax.select(p + 1 == SEQ, 0, p + 1)

        p0 = lax.rem(row0, SEQ)
        lax.fori_loop(0, CS, add_pos, p0, unroll=False)
        pltpu.sync_copy(rows_v, out.at[pl.ds(row0, CS)])
        return _

    lax.fori_loop(0, N_CHUNKS, chunk, 0, unroll=False)


@jax.jit
def _encode(table, toks2d, pos2d):
    kern = functools.partial(
        pl.kernel,
        out_type=jax.ShapeDtypeStruct((N, FEAT), jnp.float32),
        mesh=plsc.VectorSubcoreMesh(core_axis_name="c", subcore_axis_name="s"),
        scratch_types=[
            pltpu.VMEM((N_STREAMS, 128), jnp.int32),
            pltpu.VMEM((CS, FEAT), jnp.float32),
            pltpu.VMEM((SEQ, FEAT), jnp.float32),
            pltpu.SemaphoreType.DMA,
        ],
        compiler_params=pltpu.CompilerParams(use_tc_tiling_on_sc=False),
    )(_body)
    return kern(table, toks2d, pos2d)


def kernel(tokens, embed_table, pos_embedding):
    toks2d = tokens.astype(jnp.int32).reshape(N // 128, 128)
    pos2d = pos_embedding[0, :SEQ, :]
    out = _encode(embed_table, toks2d, pos2d)
    return out.reshape(BATCH, SEQ, FEAT)


# speedup vs baseline: 1.3804x; 1.3804x over previous
import functools

import jax
import jax.numpy as jnp
from jax import lax
from jax.experimental import pallas as pl
from jax.experimental.pallas import tpu as pltpu
from jax.experimental.pallas import tpu_sc as plsc

NC = 2
NS = 16
L = 16
NW = NC * NS

BATCH = 4096
SEQ = 200
POS_ROWS = 256
FEAT = 64
SEQ_PER_W = BATCH // NW
S = 2
NCH = SEQ_PER_W // S
NPAIR = NCH // 2


def _body(table, toks, pos, out, idx_v, pos_v, b0, b1, gs0, gs1, ws0, ws1):
    wid = lax.axis_index("s") * NC + lax.axis_index("c")
    batch0 = wid * SEQ_PER_W
    pltpu.sync_copy(toks.at[pl.ds(batch0, SEQ_PER_W)], idx_v)
    pltpu.sync_copy(pos.at[0], pos_v)

    def issue_gather(chunk, buf, sem):
        for s in range(S):
            pltpu.async_copy(table.at[idx_v.at[chunk * S + s]], buf.at[s], sem)

    def wait_gather(chunk, buf, sem):
        for s in range(S):
            pltpu.make_async_copy(
                table.at[idx_v.at[chunk * S + s]], buf.at[s], sem).wait()

    def issue_write(chunk, buf, sem):
        pltpu.async_copy(buf, out.at[pl.ds(batch0 + chunk * S, S)], sem)

    def wait_write(buf, sem):
        pltpu.make_async_copy(buf, out.at[pl.ds(batch0, S)], sem).wait()

    def add_pos(buf):
        def row(r, _):
            pv = [pos_v[r, pl.ds(j * L, L)] for j in range(FEAT // L)]
            for s in range(S):
                for j in range(FEAT // L):
                    sl = pl.ds(j * L, L)
                    buf[s, r, sl] = buf[s, r, sl] + pv[j]
            return _
        lax.fori_loop(0, SEQ, row, 0, unroll=False)

    issue_gather(0, b0, gs0)

    def step(i, _):
        a = 2 * i

        @pl.when(i > 0)
        def _w1():
            wait_write(b1, ws1)

        issue_gather(a + 1, b1, gs1)
        wait_gather(a, b0, gs0)
        add_pos(b0)
        issue_write(a, b0, ws0)
        wait_gather(a + 1, b1, gs1)
        add_pos(b1)

        @pl.when(i < NPAIR - 1)
        def _n0():
            wait_write(b0, ws0)
            issue_gather(a + 2, b0, gs0)

        issue_write(a + 1, b1, ws1)
        return _

    lax.fori_loop(0, NPAIR, step, 0, unroll=False)
    wait_write(b0, ws0)
    wait_write(b1, ws1)


@jax.jit
def _encode(tokens, table, pos):
    kern = functools.partial(
        pl.kernel,
        out_type=jax.ShapeDtypeStruct((BATCH, SEQ, FEAT), jnp.float32),
        mesh=plsc.VectorSubcoreMesh(core_axis_name="c", subcore_axis_name="s"),
        scratch_types=[
            pltpu.VMEM((SEQ_PER_W, SEQ), jnp.int32),
            pltpu.VMEM((POS_ROWS, FEAT), jnp.float32),
            pltpu.VMEM((S, SEQ, FEAT), jnp.float32),
            pltpu.VMEM((S, SEQ, FEAT), jnp.float32),
            pltpu.SemaphoreType.DMA,
            pltpu.SemaphoreType.DMA,
            pltpu.SemaphoreType.DMA,
            pltpu.SemaphoreType.DMA,
        ],
        compiler_params=pltpu.CompilerParams(use_tc_tiling_on_sc=False),
    )(_body)
    return kern(table, tokens, pos)


def kernel(tokens, embed_table, pos_embedding):
    return _encode(tokens.astype(jnp.int32), embed_table, pos_embedding)
